# baseline (speedup 1.0000x reference)
import functools

import jax
import jax.numpy as jnp
from jax import lax
from jax.experimental import pallas as pl
from jax.experimental.pallas import tpu as pltpu
from jax.experimental.pallas import tpu_sc as plsc

_B, _C, _D, _P = 32, 128, 128, 64
_L = 16
_NPG = _P // _L
_NDG = _D // _L
_PARAMS = _D + 2 + 14


def _round_bf16(v):
    u = plsc.bitcast(v, jnp.int32)
    r = u + jnp.int32(0x7FFF) + ((u >> 16) & 1)
    r = r & jnp.int32(-65536)
    return plsc.bitcast(r, jnp.float32)


def _sc_body(nc, x_hbm, par_hbm, out_hbm, tile_v, par_v, ts_v, rank_v, obuf_v):
    b = lax.axis_index("s") * nc + lax.axis_index("c")
    pltpu.sync_copy(par_hbm, par_v)
    tail = par_v[pl.ds(_D, _L)]
    ci = tail[1].astype(jnp.int32)
    pltpu.sync_copy(x_hbm.at[b, ci], tile_v)
    bias = tail[0]
    iota = lax.broadcasted_iota(jnp.int32, (_L,), 0)

    def score_body(d, accs):
        w = par_v[pl.ds(d, _L)][0]
        return tuple(
            accs[g] + w * _round_bf16(tile_v[d, pl.ds(g * _L, _L)])
            for g in range(_NPG)
        )

    zero = jnp.zeros((_L,), jnp.float32)
    accs = lax.fori_loop(0, _D, score_body, (zero,) * _NPG)
    ts = [(a + bias) + jnp.float32(1000000.0) for a in accs]
    for g in range(_NPG):
        ts_v[pl.ds(g * _L, _L)] = ts[g]

    pidx = [iota + g * _L for g in range(_NPG)]

    def rank_body(q, rs):
        tq = ts_v[pl.ds(q, _L)][0]
        return tuple(
            rs[g]
            + jnp.where((tq > ts[g]) | ((tq == ts[g]) & (q < pidx[g])), 1, 0)
            for g in range(_NPG)
        )

    izero = jnp.zeros((_L,), jnp.int32)
    rs = lax.fori_loop(0, _P, rank_body, (izero,) * _NPG)
    for g in range(_NPG):
        rank_v[pl.ds(g * _L, _L)] = rs[g]

    didx = [iota + j * _L for j in range(_NDG)]

    def gather_body(p, carry):
        rp = rank_v[pl.ds(p, _L)][0]
        pv = jnp.broadcast_to(p, (_L,)).astype(jnp.int32)
        for j in range(_NDG):
            col = plsc.load_gather(tile_v, [didx[j], pv])
            obuf_v[rp, pl.ds(j * _L, _L)] = col
        return carry

    lax.fori_loop(0, _P, gather_body, 0)
    pltpu.sync_copy(obuf_v, out_hbm.at[b])


def _make_call(interpret=False):
    nc, ns = 2, 16
    mesh = plsc.VectorSubcoreMesh(
        core_axis_name="c", subcore_axis_name="s", num_cores=nc, num_subcores=ns
    )
    return pl.kernel(
        functools.partial(_sc_body, nc),
        out_type=jax.ShapeDtypeStruct((_B, _P, _D), jnp.float32),
        mesh=mesh,
        scratch_types=[
            pltpu.VMEM((_D, _P), jnp.float32),
            pltpu.VMEM((_PARAMS,), jnp.float32),
            pltpu.VMEM((_P + _L,), jnp.float32),
            pltpu.VMEM((_P + _L,), jnp.int32),
            pltpu.VMEM((_P, _D), jnp.float32),
        ],
        compiler_params=pltpu.CompilerParams(needs_layout_passes=False),
        interpret=interpret,
        name="patch_selector_sc",
    )


def kernel(x, channel_idx, W, b):
    u = lax.bitcast_convert_type(W.reshape(-1).astype(jnp.float32), jnp.int32)
    u = (u + jnp.int32(0x7FFF) + ((u >> 16) & 1)) & jnp.int32(-65536)
    wr = lax.bitcast_convert_type(u, jnp.float32)
    ci = jnp.asarray(channel_idx, jnp.float32).reshape(1)
    params = jnp.concatenate([
        wr,
        jnp.asarray(b, jnp.float32).reshape(-1),
        ci,
        jnp.zeros(_PARAMS - _D - 2, jnp.float32),
    ])
    return _make_call()(x, params)


# speedup vs baseline: 6.2032x; 6.2032x over previous
import functools

import jax
import jax.numpy as jnp
from jax import lax
from jax.experimental import pallas as pl
from jax.experimental.pallas import tpu as pltpu
from jax.experimental.pallas import tpu_sc as plsc

_B, _C, _D, _P = 32, 128, 128, 64
_L = 16
_NPG = _P // _L
_NDG = _D // _L
_PARAMS = _D + 2 + 14


def _round_bf16(v):
    u = plsc.bitcast(v, jnp.int32)
    r = u + jnp.int32(0x7FFF) + ((u >> 16) & 1)
    r = r & jnp.int32(-65536)
    return plsc.bitcast(r, jnp.float32)


def _sc_body(nc, x_hbm, par_hbm, out_hbm, tile_v, par_v, ts_v, rank_v, obuf_v):
    b = lax.axis_index("s") * nc + lax.axis_index("c")
    pltpu.sync_copy(par_hbm, par_v)
    tail = par_v[pl.ds(_D, _L)]
    pltpu.sync_copy(x_hbm.at[b], tile_v)
    bias = tail[0]
    iota = lax.broadcasted_iota(jnp.int32, (_L,), 0)

    def score_body(d, accs):
        w = par_v[pl.ds(d, _L)][0]
        return tuple(
            accs[g] + w * _round_bf16(tile_v[d, pl.ds(g * _L, _L)])
            for g in range(_NPG)
        )

    zero = jnp.zeros((_L,), jnp.float32)
    accs = lax.fori_loop(0, _D, score_body, (zero,) * _NPG)
    ts = [(a + bias) + jnp.float32(1000000.0) for a in accs]
    for g in range(_NPG):
        ts_v[pl.ds(g * _L, _L)] = ts[g]

    pidx = [iota + g * _L for g in range(_NPG)]

    def rank_body(q, rs):
        tq = ts_v[pl.ds(q, _L)][0]
        return tuple(
            rs[g]
            + jnp.where((tq > ts[g]) | ((tq == ts[g]) & (q < pidx[g])), 1, 0)
            for g in range(_NPG)
        )

    izero = jnp.zeros((_L,), jnp.int32)
    rs = lax.fori_loop(0, _P, rank_body, (izero,) * _NPG)
    for g in range(_NPG):
        rank_v[pl.ds(g * _L, _L)] = rs[g]

    didx = [iota + j * _L for j in range(_NDG)]

    def gather_body(p, carry):
        rp = rank_v[pl.ds(p, _L)][0]
        pv = jnp.broadcast_to(p, (_L,)).astype(jnp.int32)
        for j in range(_NDG):
            col = plsc.load_gather(tile_v, [didx[j], pv])
            obuf_v[rp, pl.ds(j * _L, _L)] = col
        return carry

    lax.fori_loop(0, _P, gather_body, 0)
    pltpu.sync_copy(obuf_v, out_hbm.at[b])


def _make_call(interpret=False):
    nc, ns = 2, 16
    mesh = plsc.VectorSubcoreMesh(
        core_axis_name="c", subcore_axis_name="s", num_cores=nc, num_subcores=ns
    )
    return pl.kernel(
        functools.partial(_sc_body, nc),
        out_type=jax.ShapeDtypeStruct((_B, _P, _D), jnp.float32),
        mesh=mesh,
        scratch_types=[
            pltpu.VMEM((_D, _P), jnp.float32),
            pltpu.VMEM((_PARAMS,), jnp.float32),
            pltpu.VMEM((_P + _L,), jnp.float32),
            pltpu.VMEM((_P + _L,), jnp.int32),
            pltpu.VMEM((_P, _D), jnp.float32),
        ],
        compiler_params=pltpu.CompilerParams(needs_layout_passes=False),
        interpret=interpret,
        name="patch_selector_sc",
    )


def kernel(x, channel_idx, W, b):
    u = lax.bitcast_convert_type(W.reshape(-1).astype(jnp.float32), jnp.int32)
    u = (u + jnp.int32(0x7FFF) + ((u >> 16) & 1)) & jnp.int32(-65536)
    wr = lax.bitcast_convert_type(u, jnp.float32)
    ci = jnp.asarray(channel_idx, jnp.float32).reshape(1)
    params = jnp.concatenate([
        wr,
        jnp.asarray(b, jnp.float32).reshape(-1),
        ci,
        jnp.zeros(_PARAMS - _D - 2, jnp.float32),
    ])
    ci_i = jnp.asarray(channel_idx, jnp.int32)
    xc = lax.squeeze(lax.dynamic_slice_in_dim(x, ci_i, 1, axis=1), (1,))
    return _make_call()(xc, params)
